# initial kernel scaffold (unmeasured)
import jax
import jax.numpy as jnp
from jax import lax
from jax.experimental import pallas as pl
from jax.experimental.pallas import tpu as pltpu


def kernel(
    x,
):
    def body(*refs):
        pass

    out_shape = jax.ShapeDtypeStruct(..., jnp.float32)
    return pl.pallas_call(body, out_shape=out_shape)(...)


# baseline (device time: 15977 ns/iter reference)
import jax
import jax.numpy as jnp
from jax import lax
from jax.experimental import pallas as pl
from jax.experimental.pallas import tpu as pltpu

N_DEV = 8
G = 8


def kernel(x):
    m, n = x.shape

    def body(x_ref, out_ref, comm_ref, send_sems, recv_sems):
        my = lax.axis_index("i")

        partial = jnp.sum(x_ref[...].reshape(G, m // G, n), axis=1)
        comm_ref[my, :, :] = partial

        rdmas = []
        for k in range(1, N_DEV):
            dst = lax.rem(my + k, N_DEV)
            rdma = pltpu.make_async_remote_copy(
                src_ref=comm_ref.at[my],
                dst_ref=comm_ref.at[my],
                send_sem=send_sems.at[k],
                recv_sem=recv_sems.at[k],
                device_id=(dst,),
                device_id_type=pl.DeviceIdType.MESH,
            )
            rdma.start()
            rdmas.append(rdma)

        for k in range(1, N_DEV):
            src = lax.rem(my - k + N_DEV, N_DEV)
            recv = pltpu.make_async_remote_copy(
                src_ref=comm_ref.at[src],
                dst_ref=comm_ref.at[src],
                send_sem=send_sems.at[k],
                recv_sem=recv_sems.at[k],
                device_id=(src,),
                device_id_type=pl.DeviceIdType.MESH,
            )
            recv.wait_recv()

        for rdma in rdmas:
            rdma.wait_send()

        out_ref[...] = jnp.sum(
            comm_ref[...].reshape(N_DEV * G, n), axis=0, keepdims=True
        )

    return pl.pallas_call(
        body,
        out_shape=jax.ShapeDtypeStruct((1, n), jnp.float32),
        in_specs=[pl.BlockSpec(memory_space=pltpu.VMEM)],
        out_specs=pl.BlockSpec(memory_space=pltpu.VMEM),
        scratch_shapes=[
            pltpu.VMEM((N_DEV, G, n), jnp.float32),
            pltpu.SemaphoreType.DMA((N_DEV,)),
            pltpu.SemaphoreType.DMA((N_DEV,)),
        ],
    )(x)
